# baseline (speedup 1.0000x reference)
import dataclasses
import functools

import jax
import jax.numpy as jnp
from jax import lax
from jax.experimental import pallas as pl
from jax.experimental.pallas import tpu as pltpu
from jax.experimental.pallas import tpu_sc as plsc

_TOKENS_PER_BLOCK = 2048
_N_EXP = 8


def _logits_block(x_ref, w_ref, b_ref, lt_ref):
    logits = jax.lax.dot_general(
        x_ref[...], w_ref[...], (((1,), (1,)), ((), ())),
        preferred_element_type=jnp.float32,
        precision=jax.lax.Precision.DEFAULT,
    )
    lt_ref[...] = logits.T + b_ref[...].T


def _tc_logits(x, W, b2, T, D):
    tb = _TOKENS_PER_BLOCK
    return pl.pallas_call(
        _logits_block,
        grid=(T // tb,),
        in_specs=[
            pl.BlockSpec((tb, D), lambda i: (i, 0)),
            pl.BlockSpec((_N_EXP, D), lambda i: (0, 0)),
            pl.BlockSpec((1, _N_EXP), lambda i: (0, 0)),
        ],
        out_specs=pl.BlockSpec((_N_EXP, tb), lambda i: (0, i)),
        out_shape=jax.ShapeDtypeStruct((_N_EXP, T), jnp.float32),
    )(x, W, b2)


def _route_sc_kernel(lt_hbm, tw_hbm, ti_hbm, part_hbm,
                     l_v, tw_v, ti_v, p_v):
    ch = l_v.shape[1]
    wid = lax.axis_index("s") * 2 + lax.axis_index("c")
    base = wid * ch
    pltpu.sync_copy(lt_hbm.at[:, pl.ds(base, ch)], l_v)

    iota16 = lax.iota(jnp.int32, 16)
    zero = jnp.zeros((16,), jnp.float32)

    def body(j, carry):
        a1, a2, aw = carry
        sl = pl.ds(j * 16, 16)
        ls = [l_v[e, sl] for e in range(_N_EXP)]
        m = ls[0]
        for e in range(1, _N_EXP):
            m = jnp.maximum(m, ls[e])
        es = [jnp.exp(ls[e] - m) for e in range(_N_EXP)]
        z = es[0]
        for e in range(1, _N_EXP):
            z = z + es[e]
        v1 = es[0]
        i1 = jnp.zeros((16,), jnp.int32)
        for e in range(1, _N_EXP):
            gt = es[e] > v1
            v1 = jnp.where(gt, es[e], v1)
            i1 = jnp.where(gt, e, i1)
        ms = [jnp.where(i1 == e, -jnp.inf, es[e]) for e in range(_N_EXP)]
        v2 = ms[0]
        i2 = jnp.zeros((16,), jnp.int32)
        for e in range(1, _N_EXP):
            gt = ms[e] > v2
            v2 = jnp.where(gt, ms[e], v2)
            i2 = jnp.where(gt, e, i2)
        inv = 1.0 / (v1 + v2 + 1e-8 * z)
        g1 = v1 * inv
        g2 = v2 * inv
        pos = j * 32 + iota16 * 2
        plsc.store_scatter(tw_v, [pos], g1)
        plsc.store_scatter(tw_v, [pos + 1], g2)
        plsc.store_scatter(ti_v, [pos], i1)
        plsc.store_scatter(ti_v, [pos + 1], i2)
        return (a1 + i1.astype(jnp.float32), a2 + i2.astype(jnp.float32),
                aw + g1 + g2)

    a1, a2, aw = lax.fori_loop(0, ch // 16, body, (zero, zero, zero))
    p_v[0, :] = a1
    p_v[1, :] = a2
    p_v[2, :] = aw
    pltpu.sync_copy(tw_v, tw_hbm.at[pl.ds(2 * base, 2 * ch)])
    pltpu.sync_copy(ti_v, ti_hbm.at[pl.ds(2 * base, 2 * ch)])
    pltpu.sync_copy(p_v, part_hbm.at[wid])


def kernel(hidden_states, W, b):
    B, S, D = hidden_states.shape
    T = B * S
    x = hidden_states.reshape(T, D)
    b2 = b.reshape(1, _N_EXP)
    lt = _tc_logits(x, W, b2, T, D)

    mesh = plsc.VectorSubcoreMesh(core_axis_name="c", subcore_axis_name="s")
    ch = T // 32
    cp = pltpu.CompilerParams()
    if "needs_layout_passes" in pltpu.CompilerParams.__dataclass_fields__:
        cp = dataclasses.replace(cp, needs_layout_passes=False)
    route = pl.kernel(
        _route_sc_kernel,
        mesh=mesh,
        compiler_params=cp,
        out_type=[
            jax.ShapeDtypeStruct((2 * T,), jnp.float32),
            jax.ShapeDtypeStruct((2 * T,), jnp.int32),
            jax.ShapeDtypeStruct((32, 3, 16), jnp.float32),
        ],
        scratch_types=[
            pltpu.VMEM((_N_EXP, ch), jnp.float32),
            pltpu.VMEM((2 * ch,), jnp.float32),
            pltpu.VMEM((2 * ch,), jnp.int32),
            pltpu.VMEM((3, 16), jnp.float32),
        ],
    )
    twf, tif, parts = route(lt)
    top_k_weights = twf.reshape(B, S, 2)
    top_k_indices = tif.reshape(B, S, 2)
    sums = jnp.sum(parts, axis=(0, 2))
    expert_usage = sums[:2] / T
    avg_router_confidence = sums[2] / (T * 2)
    return (top_k_weights, top_k_indices, expert_usage, avg_router_confidence)


# speedup vs baseline: 2.5530x; 2.5530x over previous
import jax
import jax.numpy as jnp
from jax.experimental import pallas as pl

_TOKENS_PER_BLOCK = 2048


def _router_block(x_ref, w_ref, b_ref, tw_ref, ti_ref, acc_ref):
    x = x_ref[...]
    w = w_ref[...]
    logits = jax.lax.dot_general(
        x, w, (((1,), (1,)), ((), ())),
        preferred_element_type=jnp.float32,
        precision=jax.lax.Precision.DEFAULT,
    )
    lt = logits.T + b_ref[...].T

    m = jnp.max(lt, axis=0, keepdims=True)
    e = jnp.exp(lt - m)
    z = jnp.sum(e, axis=0, keepdims=True)

    n_e = e.shape[0]
    idx = jax.lax.broadcasted_iota(jnp.int32, e.shape, 0)
    w1 = jnp.max(e, axis=0, keepdims=True)
    i1 = jnp.min(jnp.where(e == w1, idx, n_e), axis=0, keepdims=True)
    e2m = jnp.where(idx == i1, -jnp.inf, e)
    w2 = jnp.max(e2m, axis=0, keepdims=True)
    i2 = jnp.min(jnp.where(e2m == w2, idx, n_e), axis=0, keepdims=True)

    inv = 1.0 / (w1 + w2 + 1e-8 * z)
    g1 = w1 * inv
    g2 = w2 * inv
    tw_ref[...] = jnp.concatenate([g1, g2], axis=0)
    ti_ref[...] = jnp.concatenate([i1, i2], axis=0)

    s_i1 = jnp.sum(i1.astype(jnp.float32))
    s_i2 = jnp.sum(i2.astype(jnp.float32))
    s_w = jnp.sum(g1) + jnp.sum(g2)
    lane = jax.lax.broadcasted_iota(jnp.int32, acc_ref.shape, 1)
    part = (jnp.where(lane == 0, s_i1, 0.0)
            + jnp.where(lane == 1, s_i2, 0.0)
            + jnp.where(lane == 2, s_w, 0.0))

    @pl.when(pl.program_id(0) == 0)
    def _():
        acc_ref[...] = part

    @pl.when(pl.program_id(0) != 0)
    def _():
        acc_ref[...] = acc_ref[...] + part


def kernel(hidden_states, W, b):
    B, S, D = hidden_states.shape
    T = B * S
    n_e = W.shape[0]
    x = hidden_states.reshape(T, D)
    b2 = b.reshape(1, n_e)
    tb = _TOKENS_PER_BLOCK
    twt, tit, acc = pl.pallas_call(
        _router_block,
        grid=(T // tb,),
        in_specs=[
            pl.BlockSpec((tb, D), lambda i: (i, 0)),
            pl.BlockSpec((n_e, D), lambda i: (0, 0)),
            pl.BlockSpec((1, n_e), lambda i: (0, 0)),
        ],
        out_specs=[
            pl.BlockSpec((2, tb), lambda i: (0, i)),
            pl.BlockSpec((2, tb), lambda i: (0, i)),
            pl.BlockSpec((1, 128), lambda i: (0, 0)),
        ],
        out_shape=[
            jax.ShapeDtypeStruct((2, T), jnp.float32),
            jax.ShapeDtypeStruct((2, T), jnp.int32),
            jax.ShapeDtypeStruct((1, 128), jnp.float32),
        ],
    )(x, W, b2)
    top_k_weights = twt.T.reshape(B, S, 2)
    top_k_indices = tit.T.reshape(B, S, 2)
    expert_usage = acc[0, :2] / T
    avg_router_confidence = acc[0, 2] / (T * 2)
    return (top_k_weights, top_k_indices, expert_usage, avg_router_confidence)
